# Initial kernel scaffold; baseline (speedup 1.0000x reference)
import jax
import jax.numpy as jnp
from jax.experimental import pallas as pl


def kernel(x_sentance, x_token, x_token_ori, edge_index, W1, b1, W2, b2):
    raise NotImplementedError("write your pallas kernel here")


# speedup vs baseline: 13.6299x; 13.6299x over previous
import functools

import jax
import jax.numpy as jnp
from jax import lax
from jax.experimental import pallas as pl
from jax.experimental.pallas import tpu as pltpu
from jax.experimental.pallas import tpu_sc as plsc

N = 10000
E = 320000
D = 128
T = 20

NC = 2
NS = 16
ROWS_PER_TILE = N // NS
K = 125
BLOCKS_A = E // (NC * NS) // K
BLOCKS_C = E // NS // K
CHUNK_C = 32

def _sc_degree_body(dst_hbm, zeros_hbm, ones_hbm, out_hbm, acc, onesv, dstv):
    c = lax.axis_index("c")
    s = lax.axis_index("s")
    r0 = s * ROWS_PER_TILE
    pltpu.sync_copy(zeros_hbm, acc.at[pl.ds(r0, ROWS_PER_TILE)])
    pltpu.sync_copy(ones_hbm, onesv)
    pltpu.sync_copy(dst_hbm.at[c, s], dstv)
    plsc.subcore_barrier()

    def step(j, carry):
        pltpu.sync_copy(onesv, acc.at[dstv.at[j]], add=True)
        return carry

    lax.fori_loop(0, BLOCKS_A, step, 0)
    plsc.subcore_barrier()
    pltpu.sync_copy(acc.at[pl.ds(r0, ROWS_PER_TILE)], out_hbm.at[c, s])


def _sc_gather_scatter_body(g_hbm, src_hbm, dst_hbm, zeros_hbm, out_hbm,
                            acc, rows, srcv, dstv, sem):
    c = lax.axis_index("c")
    s = lax.axis_index("s")
    r0 = s * ROWS_PER_TILE
    pltpu.sync_copy(zeros_hbm, acc.at[pl.ds(r0, ROWS_PER_TILE)])
    plsc.subcore_barrier()

    def chunk(q, carry):
        pltpu.sync_copy(src_hbm.at[c, s, pl.ds(q * CHUNK_C, CHUNK_C)], srcv)
        pltpu.sync_copy(dst_hbm.at[s, pl.ds(q * CHUNK_C, CHUNK_C)], dstv)

        def step(j, carry2):
            pltpu.async_copy(g_hbm.at[srcv.at[j]], rows, sem).wait()
            pltpu.sync_copy(rows, acc.at[dstv.at[j]], add=True)
            return carry2

        return lax.fori_loop(0, CHUNK_C, step, carry)

    lax.fori_loop(0, BLOCKS_C // CHUNK_C, chunk, 0)
    plsc.subcore_barrier()
    pltpu.sync_copy(acc.at[pl.ds(r0, ROWS_PER_TILE)], out_hbm.at[c, s])


@functools.lru_cache(maxsize=None)
def _sc_kernels():
    mesh = plsc.VectorSubcoreMesh(core_axis_name="c", subcore_axis_name="s")
    sc_degree = pl.kernel(
        _sc_degree_body,
        mesh=mesh,
        out_type=jax.ShapeDtypeStruct((NC, NS, ROWS_PER_TILE, D),
                                      jnp.float32),
        scratch_types=[
            pltpu.VMEM_SHARED((N, D), jnp.float32),
            pltpu.VMEM((K, D), jnp.float32),
            pltpu.VMEM((BLOCKS_A, K), jnp.int32),
        ],
    )
    sc_gather_scatter = pl.kernel(
        _sc_gather_scatter_body,
        mesh=mesh,
        out_type=jax.ShapeDtypeStruct((NC, NS, ROWS_PER_TILE, D),
                                      jnp.float32),
        scratch_types=[
            pltpu.VMEM_SHARED((N, D), jnp.float32),
            pltpu.VMEM((K, D), jnp.float32),
            pltpu.VMEM((CHUNK_C, K), jnp.int32),
            pltpu.VMEM((CHUNK_C, K), jnp.int32),
            pltpu.SemaphoreType.DMA,
        ],
    )
    return sc_degree, sc_gather_scatter


def _prep_body(xs_ref, xt_ref, w1_ref, w2_ref, degp_ref, g_ref):
    deg = degp_ref[0][:, 0:1] + degp_ref[1][:, 0:1] + 1.0
    dinv = lax.rsqrt(deg)
    x = xs_ref[...]
    h1 = lax.dot_general(x, w1_ref[...], (((1,), (1,)), ((), ())),
                         preferred_element_type=jnp.float32)
    h2 = lax.dot_general(x + xt_ref[...], w2_ref[...], (((1,), (1,)), ((), ())),
                         preferred_element_type=jnp.float32)
    g_ref[0] = h1 * dinv
    g_ref[1] = h2 * dinv


def _tc_prepare(xs, xt, W1, W2, degp):
    bn = 1000
    grid = (N // bn,)
    return pl.pallas_call(
        _prep_body,
        grid=grid,
        in_specs=[
            pl.BlockSpec((bn, D), lambda i: (i, 0)),
            pl.BlockSpec((bn, D), lambda i: (i, 0)),
            pl.BlockSpec((D, D), lambda i: (0, 0)),
            pl.BlockSpec((D, D), lambda i: (0, 0)),
            pl.BlockSpec((NC, bn, D), lambda i: (0, i, 0)),
        ],
        out_specs=pl.BlockSpec((NC, bn, D), lambda i: (0, i, 0)),
        out_shape=jax.ShapeDtypeStruct((NC, N, D), jnp.float32),
    )(xs, xt, W1, W2, degp)


def _final_body(acc_ref, g_ref, degp_ref, xs_ref, xto_ref, b1_ref, b2_ref,
                embs_ref, etf_ref, cs_ref, ct_ref, *, ngrid):
    deg = degp_ref[0][:, 0:1] + degp_ref[1][:, 0:1] + 1.0
    dinv = lax.rsqrt(deg)
    xs = xs_ref[...]
    emb_s = dinv * (acc_ref[0] + g_ref[0]) + b1_ref[...] + xs
    etn = dinv * (acc_ref[1] + g_ref[1]) + b2_ref[...] + xs
    embs_ref[...] = emb_s
    xto = xto_ref[...]
    etf_ref[...] = xto + etn[:, None, :]

    cs_part = jnp.sum(emb_s, axis=0, keepdims=True)
    ct_part = (jnp.sum(xto, axis=(0, 1))[None, :] * (1.0 / T)
               + jnp.sum(etn, axis=0, keepdims=True))
    i = pl.program_id(0)

    @pl.when(i == 0)
    def _():
        cs_ref[...] = jnp.zeros_like(cs_ref)
        ct_ref[...] = jnp.zeros_like(ct_ref)

    cs_ref[...] += cs_part
    ct_ref[...] += ct_part

    @pl.when(i == ngrid - 1)
    def _():
        cs_ref[...] *= 1.0 / N
        ct_ref[...] *= 1.0 / N


def _tc_finalize(acc, g, degp, xs, xto, b1, b2):
    bn = 200
    ngrid = N // bn
    return pl.pallas_call(
        functools.partial(_final_body, ngrid=ngrid),
        grid=(ngrid,),
        in_specs=[
            pl.BlockSpec((NC, bn, D), lambda i: (0, i, 0)),
            pl.BlockSpec((NC, bn, D), lambda i: (0, i, 0)),
            pl.BlockSpec((NC, bn, D), lambda i: (0, i, 0)),
            pl.BlockSpec((bn, D), lambda i: (i, 0)),
            pl.BlockSpec((bn, T, D), lambda i: (i, 0, 0)),
            pl.BlockSpec((1, D), lambda i: (0, 0)),
            pl.BlockSpec((1, D), lambda i: (0, 0)),
        ],
        out_specs=[
            pl.BlockSpec((bn, D), lambda i: (i, 0)),
            pl.BlockSpec((bn, T, D), lambda i: (i, 0, 0)),
            pl.BlockSpec((1, D), lambda i: (0, 0)),
            pl.BlockSpec((1, D), lambda i: (0, 0)),
        ],
        out_shape=[
            jax.ShapeDtypeStruct((N, D), jnp.float32),
            jax.ShapeDtypeStruct((N, T, D), jnp.float32),
            jax.ShapeDtypeStruct((1, D), jnp.float32),
            jax.ShapeDtypeStruct((1, D), jnp.float32),
        ],
    )(acc, g, degp, xs, xto, b1, b2)


def kernel(x_sentance, x_token, x_token_ori, edge_index, W1, b1, W2, b2):
    src = edge_index[0]
    dst = edge_index[1]

    dstA = dst.reshape(NC, NS, BLOCKS_A, K)
    dstC = dst.reshape(NS, BLOCKS_C, K)
    srcC = src.reshape(NS, BLOCKS_C, K)
    srcC2 = jnp.stack([srcC, srcC + N])

    zerosD = jnp.zeros((ROWS_PER_TILE, D), jnp.float32)
    onesD = jnp.ones((K, D), jnp.float32)

    sc_degree, sc_gather_scatter = _sc_kernels()
    degp = sc_degree(dstA, zerosD, onesD).reshape(NC, N, D)
    g = _tc_prepare(x_sentance, x_token, W1, W2, degp)
    acc = sc_gather_scatter(g.reshape(NC * N, D), srcC2, dstC,
                            zerosD).reshape(NC, N, D)

    emb_s, etf, cs, ct = _tc_finalize(
        acc, g, degp, x_sentance, x_token_ori,
        b1.reshape(1, D), b2.reshape(1, D))
    return emb_s, etf, cs.reshape(D), ct.reshape(D)


# speedup vs baseline: 15.4760x; 1.1354x over previous
import functools

import jax
import jax.numpy as jnp
from jax import lax
from jax.experimental import pallas as pl
from jax.experimental.pallas import tpu as pltpu
from jax.experimental.pallas import tpu_sc as plsc

N = 10000
E = 320000
D = 128
T = 20

NC = 2
NS = 16
ROWS_PER_TILE = N // NS
K = 125
BLOCKS_A = E // (NC * NS) // K
BLOCKS_C = E // NS // K
CHUNK_C = 32

def _sc_degree_body(dst_hbm, zeros_hbm, ones_hbm, out_hbm, acc, onesv, dstv):
    c = lax.axis_index("c")
    s = lax.axis_index("s")
    r0 = s * ROWS_PER_TILE
    pltpu.sync_copy(zeros_hbm, acc.at[pl.ds(r0, ROWS_PER_TILE)])
    pltpu.sync_copy(ones_hbm, onesv)
    pltpu.sync_copy(dst_hbm.at[c, s], dstv)
    plsc.subcore_barrier()

    def step(j, carry):
        pltpu.sync_copy(onesv, acc.at[dstv.at[j]], add=True)
        return carry

    lax.fori_loop(0, BLOCKS_A, step, 0)
    plsc.subcore_barrier()
    pltpu.sync_copy(acc.at[pl.ds(r0, ROWS_PER_TILE)], out_hbm.at[c, s])


def _sc_gather_scatter_body(g_hbm, src_hbm, dst_hbm, zeros_hbm, out_hbm,
                            acc, rows0, rows1, srcv, dstv,
                            gsem0, gsem1, ssem0, ssem1):
    c = lax.axis_index("c")
    s = lax.axis_index("s")
    r0 = s * ROWS_PER_TILE
    pltpu.sync_copy(zeros_hbm, acc.at[pl.ds(r0, ROWS_PER_TILE)])
    plsc.subcore_barrier()

    rows = (rows0, rows1)
    gsem = (gsem0, gsem1)
    ssem = (ssem0, ssem1)

    def chunk(q, carry):
        pltpu.sync_copy(src_hbm.at[c, s, pl.ds(q * CHUNK_C, CHUNK_C)], srcv)
        pltpu.sync_copy(dst_hbm.at[s, pl.ds(q * CHUNK_C, CHUNK_C)], dstv)
        hg = [None] * CHUNK_C
        hs = [None] * CHUNK_C
        hg[0] = pltpu.async_copy(g_hbm.at[srcv.at[0]], rows[0], gsem[0])
        for j in range(CHUNK_C):
            b = j % 2
            hg[j].wait()
            hs[j] = pltpu.async_copy(rows[b], acc.at[dstv.at[j]], ssem[b],
                                     add=True)
            if j + 1 < CHUNK_C:
                if j >= 1:
                    hs[j - 1].wait()
                hg[j + 1] = pltpu.async_copy(g_hbm.at[srcv.at[j + 1]],
                                             rows[1 - b], gsem[1 - b])
        hs[CHUNK_C - 2].wait()
        hs[CHUNK_C - 1].wait()
        return carry

    lax.fori_loop(0, BLOCKS_C // CHUNK_C, chunk, 0)
    plsc.subcore_barrier()
    pltpu.sync_copy(acc.at[pl.ds(r0, ROWS_PER_TILE)], out_hbm.at[c, s])


@functools.lru_cache(maxsize=None)
def _sc_kernels():
    mesh = plsc.VectorSubcoreMesh(core_axis_name="c", subcore_axis_name="s")
    sc_degree = pl.kernel(
        _sc_degree_body,
        mesh=mesh,
        out_type=jax.ShapeDtypeStruct((NC, NS, ROWS_PER_TILE, D),
                                      jnp.float32),
        scratch_types=[
            pltpu.VMEM_SHARED((N, D), jnp.float32),
            pltpu.VMEM((K, D), jnp.float32),
            pltpu.VMEM((BLOCKS_A, K), jnp.int32),
        ],
    )
    sc_gather_scatter = pl.kernel(
        _sc_gather_scatter_body,
        mesh=mesh,
        out_type=jax.ShapeDtypeStruct((NC, NS, ROWS_PER_TILE, D),
                                      jnp.float32),
        scratch_types=[
            pltpu.VMEM_SHARED((N, D), jnp.float32),
            pltpu.VMEM((K, D), jnp.float32),
            pltpu.VMEM((K, D), jnp.float32),
            pltpu.VMEM((CHUNK_C, K), jnp.int32),
            pltpu.VMEM((CHUNK_C, K), jnp.int32),
            pltpu.SemaphoreType.DMA,
            pltpu.SemaphoreType.DMA,
            pltpu.SemaphoreType.DMA,
            pltpu.SemaphoreType.DMA,
        ],
    )
    return sc_degree, sc_gather_scatter


def _prep_body(xs_ref, xt_ref, w1_ref, w2_ref, degp_ref, g_ref):
    deg = degp_ref[0][:, 0:1] + degp_ref[1][:, 0:1] + 1.0
    dinv = lax.rsqrt(deg)
    x = xs_ref[...]
    h1 = lax.dot_general(x, w1_ref[...], (((1,), (1,)), ((), ())),
                         preferred_element_type=jnp.float32)
    h2 = lax.dot_general(x + xt_ref[...], w2_ref[...], (((1,), (1,)), ((), ())),
                         preferred_element_type=jnp.float32)
    g_ref[0] = h1 * dinv
    g_ref[1] = h2 * dinv


def _tc_prepare(xs, xt, W1, W2, degp):
    bn = 1000
    grid = (N // bn,)
    return pl.pallas_call(
        _prep_body,
        grid=grid,
        in_specs=[
            pl.BlockSpec((bn, D), lambda i: (i, 0)),
            pl.BlockSpec((bn, D), lambda i: (i, 0)),
            pl.BlockSpec((D, D), lambda i: (0, 0)),
            pl.BlockSpec((D, D), lambda i: (0, 0)),
            pl.BlockSpec((NC, bn, D), lambda i: (0, i, 0)),
        ],
        out_specs=pl.BlockSpec((NC, bn, D), lambda i: (0, i, 0)),
        out_shape=jax.ShapeDtypeStruct((NC, N, D), jnp.float32),
    )(xs, xt, W1, W2, degp)


def _final_body(acc_ref, g_ref, degp_ref, xs_ref, xto_ref, b1_ref, b2_ref,
                embs_ref, etf_ref, cs_ref, ct_ref, *, ngrid):
    deg = degp_ref[0][:, 0:1] + degp_ref[1][:, 0:1] + 1.0
    dinv = lax.rsqrt(deg)
    xs = xs_ref[...]
    emb_s = dinv * (acc_ref[0] + g_ref[0]) + b1_ref[...] + xs
    etn = dinv * (acc_ref[1] + g_ref[1]) + b2_ref[...] + xs
    embs_ref[...] = emb_s
    xto = xto_ref[...]
    etf_ref[...] = xto + etn[:, None, :]

    cs_part = jnp.sum(emb_s, axis=0, keepdims=True)
    ct_part = (jnp.sum(xto, axis=(0, 1))[None, :] * (1.0 / T)
               + jnp.sum(etn, axis=0, keepdims=True))
    i = pl.program_id(0)

    @pl.when(i == 0)
    def _():
        cs_ref[...] = jnp.zeros_like(cs_ref)
        ct_ref[...] = jnp.zeros_like(ct_ref)

    cs_ref[...] += cs_part
    ct_ref[...] += ct_part

    @pl.when(i == ngrid - 1)
    def _():
        cs_ref[...] *= 1.0 / N
        ct_ref[...] *= 1.0 / N


def _tc_finalize(acc, g, degp, xs, xto, b1, b2):
    bn = 200
    ngrid = N // bn
    return pl.pallas_call(
        functools.partial(_final_body, ngrid=ngrid),
        grid=(ngrid,),
        in_specs=[
            pl.BlockSpec((NC, bn, D), lambda i: (0, i, 0)),
            pl.BlockSpec((NC, bn, D), lambda i: (0, i, 0)),
            pl.BlockSpec((NC, bn, D), lambda i: (0, i, 0)),
            pl.BlockSpec((bn, D), lambda i: (i, 0)),
            pl.BlockSpec((bn, T, D), lambda i: (i, 0, 0)),
            pl.BlockSpec((1, D), lambda i: (0, 0)),
            pl.BlockSpec((1, D), lambda i: (0, 0)),
        ],
        out_specs=[
            pl.BlockSpec((bn, D), lambda i: (i, 0)),
            pl.BlockSpec((bn, T, D), lambda i: (i, 0, 0)),
            pl.BlockSpec((1, D), lambda i: (0, 0)),
            pl.BlockSpec((1, D), lambda i: (0, 0)),
        ],
        out_shape=[
            jax.ShapeDtypeStruct((N, D), jnp.float32),
            jax.ShapeDtypeStruct((N, T, D), jnp.float32),
            jax.ShapeDtypeStruct((1, D), jnp.float32),
            jax.ShapeDtypeStruct((1, D), jnp.float32),
        ],
    )(acc, g, degp, xs, xto, b1, b2)


def kernel(x_sentance, x_token, x_token_ori, edge_index, W1, b1, W2, b2):
    src = edge_index[0]
    dst = edge_index[1]

    dstA = dst.reshape(NC, NS, BLOCKS_A, K)
    dstC = dst.reshape(NS, BLOCKS_C, K)
    srcC = src.reshape(NS, BLOCKS_C, K)
    srcC2 = jnp.stack([srcC, srcC + N])

    zerosD = jnp.zeros((ROWS_PER_TILE, D), jnp.float32)
    onesD = jnp.ones((K, D), jnp.float32)

    sc_degree, sc_gather_scatter = _sc_kernels()
    degp = sc_degree(dstA, zerosD, onesD).reshape(NC, N, D)
    g = _tc_prepare(x_sentance, x_token, W1, W2, degp)
    acc = sc_gather_scatter(g.reshape(NC * N, D), srcC2, dstC,
                            zerosD).reshape(NC, N, D)

    emb_s, etf, cs, ct = _tc_finalize(
        acc, g, degp, x_sentance, x_token_ori,
        b1.reshape(1, D), b2.reshape(1, D))
    return emb_s, etf, cs.reshape(D), ct.reshape(D)


# speedup vs baseline: 15.9149x; 1.0284x over previous
import functools

import jax
import jax.numpy as jnp
from jax import lax
from jax.experimental import pallas as pl
from jax.experimental.pallas import tpu as pltpu
from jax.experimental.pallas import tpu_sc as plsc

N = 10000
E = 320000
D = 128
T = 20

NC = 2
NS = 16
ROWS_PER_TILE = N // NS
OUT_TILES = 10
OUT_ROWS = N // OUT_TILES
K = 125
BLOCKS_A = E // (NC * NS) // K
BLOCKS_C = E // NS // K
CHUNK_C = 32

def _sc_degree_body(dst_hbm, zeros_hbm, ones_hbm, out_hbm, acc, onesv, dstv):
    c = lax.axis_index("c")
    s = lax.axis_index("s")
    r0 = s * ROWS_PER_TILE
    pltpu.sync_copy(zeros_hbm, acc.at[pl.ds(r0, ROWS_PER_TILE)])
    pltpu.sync_copy(ones_hbm, onesv)
    pltpu.sync_copy(dst_hbm.at[c, s], dstv)
    plsc.subcore_barrier()

    def step(j, carry):
        pltpu.sync_copy(onesv, acc.at[dstv.at[j]], add=True)
        return carry

    lax.fori_loop(0, BLOCKS_A, step, 0)
    plsc.subcore_barrier()

    @pl.when(s < OUT_TILES)
    def _():
        pltpu.sync_copy(acc.at[pl.ds(s * OUT_ROWS, OUT_ROWS)],
                        out_hbm.at[c, s])


def _sc_gather_scatter_body(g_hbm, src_hbm, dst_hbm, zeros_hbm, out_hbm,
                            acc, rows0, rows1, srcv, dstv,
                            gsem0, gsem1, ssem0, ssem1):
    c = lax.axis_index("c")
    s = lax.axis_index("s")
    r0 = s * ROWS_PER_TILE
    pltpu.sync_copy(zeros_hbm, acc.at[pl.ds(r0, ROWS_PER_TILE)])
    plsc.subcore_barrier()

    rows = (rows0, rows1)
    gsem = (gsem0, gsem1)
    ssem = (ssem0, ssem1)

    def chunk(q, carry):
        pltpu.sync_copy(src_hbm.at[c, s, pl.ds(q * CHUNK_C, CHUNK_C)], srcv)
        pltpu.sync_copy(dst_hbm.at[s, pl.ds(q * CHUNK_C, CHUNK_C)], dstv)
        hg = [None] * CHUNK_C
        hs = [None] * CHUNK_C
        hg[0] = pltpu.async_copy(g_hbm.at[srcv.at[0]], rows[0], gsem[0])
        for j in range(CHUNK_C):
            b = j % 2
            hg[j].wait()
            hs[j] = pltpu.async_copy(rows[b], acc.at[dstv.at[j]], ssem[b],
                                     add=True)
            if j + 1 < CHUNK_C:
                if j >= 1:
                    hs[j - 1].wait()
                hg[j + 1] = pltpu.async_copy(g_hbm.at[srcv.at[j + 1]],
                                             rows[1 - b], gsem[1 - b])
        hs[CHUNK_C - 2].wait()
        hs[CHUNK_C - 1].wait()
        return carry

    lax.fori_loop(0, BLOCKS_C // CHUNK_C, chunk, 0)
    plsc.subcore_barrier()

    @pl.when(s < OUT_TILES)
    def _():
        pltpu.sync_copy(acc.at[pl.ds(s * OUT_ROWS, OUT_ROWS)],
                        out_hbm.at[c, s])


@functools.lru_cache(maxsize=None)
def _sc_kernels():
    mesh = plsc.VectorSubcoreMesh(core_axis_name="c", subcore_axis_name="s")
    sc_degree = pl.kernel(
        _sc_degree_body,
        mesh=mesh,
        out_type=jax.ShapeDtypeStruct((NC, OUT_TILES, OUT_ROWS, D),
                                      jnp.float32),
        scratch_types=[
            pltpu.VMEM_SHARED((N, D), jnp.float32),
            pltpu.VMEM((K, D), jnp.float32),
            pltpu.VMEM((BLOCKS_A, K), jnp.int32),
        ],
    )
    sc_gather_scatter = pl.kernel(
        _sc_gather_scatter_body,
        mesh=mesh,
        out_type=jax.ShapeDtypeStruct((NC, OUT_TILES, OUT_ROWS, D),
                                      jnp.float32),
        scratch_types=[
            pltpu.VMEM_SHARED((N, D), jnp.float32),
            pltpu.VMEM((K, D), jnp.float32),
            pltpu.VMEM((K, D), jnp.float32),
            pltpu.VMEM((CHUNK_C, K), jnp.int32),
            pltpu.VMEM((CHUNK_C, K), jnp.int32),
            pltpu.SemaphoreType.DMA,
            pltpu.SemaphoreType.DMA,
            pltpu.SemaphoreType.DMA,
            pltpu.SemaphoreType.DMA,
        ],
    )
    return sc_degree, sc_gather_scatter


def _prep_body(xs_ref, xt_ref, w1_ref, w2_ref, degp_ref, g_ref):
    deg = degp_ref[0][:, 0:1] + degp_ref[1][:, 0:1] + 1.0
    dinv = lax.rsqrt(deg)
    x = xs_ref[...]
    h1 = lax.dot_general(x, w1_ref[...], (((1,), (1,)), ((), ())),
                         preferred_element_type=jnp.float32)
    h2 = lax.dot_general(x + xt_ref[...], w2_ref[...], (((1,), (1,)), ((), ())),
                         preferred_element_type=jnp.float32)
    g_ref[0] = h1 * dinv
    g_ref[1] = h2 * dinv


def _tc_prepare(xs, xt, W1, W2, degp):
    bn = 1000
    grid = (N // bn,)
    return pl.pallas_call(
        _prep_body,
        grid=grid,
        in_specs=[
            pl.BlockSpec((bn, D), lambda i: (i, 0)),
            pl.BlockSpec((bn, D), lambda i: (i, 0)),
            pl.BlockSpec((D, D), lambda i: (0, 0)),
            pl.BlockSpec((D, D), lambda i: (0, 0)),
            pl.BlockSpec((NC, bn, D), lambda i: (0, i, 0)),
        ],
        out_specs=pl.BlockSpec((NC, bn, D), lambda i: (0, i, 0)),
        out_shape=jax.ShapeDtypeStruct((NC, N, D), jnp.float32),
    )(xs, xt, W1, W2, degp)


def _final_body(acc_ref, g_ref, degp_ref, xs_ref, xto_ref, b1_ref, b2_ref,
                embs_ref, etf_ref, cs_ref, ct_ref, *, ngrid):
    deg = degp_ref[0][:, 0:1] + degp_ref[1][:, 0:1] + 1.0
    dinv = lax.rsqrt(deg)
    xs = xs_ref[...]
    emb_s = dinv * (acc_ref[0] + g_ref[0]) + b1_ref[...] + xs
    etn = dinv * (acc_ref[1] + g_ref[1]) + b2_ref[...] + xs
    embs_ref[...] = emb_s
    xto = xto_ref[...]
    etf_ref[...] = xto + etn[:, None, :]

    cs_part = jnp.sum(emb_s, axis=0, keepdims=True)
    ct_part = (jnp.sum(xto, axis=(0, 1))[None, :] * (1.0 / T)
               + jnp.sum(etn, axis=0, keepdims=True))
    i = pl.program_id(0)

    @pl.when(i == 0)
    def _():
        cs_ref[...] = jnp.zeros_like(cs_ref)
        ct_ref[...] = jnp.zeros_like(ct_ref)

    cs_ref[...] += cs_part
    ct_ref[...] += ct_part

    @pl.when(i == ngrid - 1)
    def _():
        cs_ref[...] *= 1.0 / N
        ct_ref[...] *= 1.0 / N


def _tc_finalize(acc, g, degp, xs, xto, b1, b2):
    bn = 200
    ngrid = N // bn
    return pl.pallas_call(
        functools.partial(_final_body, ngrid=ngrid),
        grid=(ngrid,),
        in_specs=[
            pl.BlockSpec((NC, bn, D), lambda i: (0, i, 0)),
            pl.BlockSpec((NC, bn, D), lambda i: (0, i, 0)),
            pl.BlockSpec((NC, bn, D), lambda i: (0, i, 0)),
            pl.BlockSpec((bn, D), lambda i: (i, 0)),
            pl.BlockSpec((bn, T, D), lambda i: (i, 0, 0)),
            pl.BlockSpec((1, D), lambda i: (0, 0)),
            pl.BlockSpec((1, D), lambda i: (0, 0)),
        ],
        out_specs=[
            pl.BlockSpec((bn, D), lambda i: (i, 0)),
            pl.BlockSpec((bn, T, D), lambda i: (i, 0, 0)),
            pl.BlockSpec((1, D), lambda i: (0, 0)),
            pl.BlockSpec((1, D), lambda i: (0, 0)),
        ],
        out_shape=[
            jax.ShapeDtypeStruct((N, D), jnp.float32),
            jax.ShapeDtypeStruct((N, T, D), jnp.float32),
            jax.ShapeDtypeStruct((1, D), jnp.float32),
            jax.ShapeDtypeStruct((1, D), jnp.float32),
        ],
    )(acc, g, degp, xs, xto, b1, b2)


def kernel(x_sentance, x_token, x_token_ori, edge_index, W1, b1, W2, b2):
    src = edge_index[0]
    dst = edge_index[1]

    dstA = dst.reshape(NC, NS, BLOCKS_A, K)
    dstC = dst.reshape(NS, BLOCKS_C, K)
    srcC = src.reshape(NS, BLOCKS_C, K)
    srcC2 = jnp.stack([srcC, srcC + N])

    zerosD = jnp.zeros((ROWS_PER_TILE, D), jnp.float32)
    onesD = jnp.ones((K, D), jnp.float32)

    sc_degree, sc_gather_scatter = _sc_kernels()
    degp = sc_degree(dstA, zerosD, onesD).reshape(NC, N, D)
    g = _tc_prepare(x_sentance, x_token, W1, W2, degp)
    acc = sc_gather_scatter(g.reshape(NC * N, D), srcC2, dstC,
                            zerosD).reshape(NC, N, D)

    emb_s, etf, cs, ct = _tc_finalize(
        acc, g, degp, x_sentance, x_token_ori,
        b1.reshape(1, D), b2.reshape(1, D))
    return emb_s, etf, cs.reshape(D), ct.reshape(D)


# speedup vs baseline: 20.1520x; 1.2662x over previous
import functools

import jax
import jax.numpy as jnp
from jax import lax
from jax.experimental import pallas as pl
from jax.experimental.pallas import tpu as pltpu
from jax.experimental.pallas import tpu_sc as plsc

N = 10000
E = 320000
D = 128
T = 20

NC = 2
NS = 16
ROWS_PER_TILE = N // NS
OUT_TILES = 10
OUT_ROWS = N // OUT_TILES
K = 125
BLOCKS_A = E // (NC * NS) // K
BLOCKS_C = E // NS // K
CHUNK_C = 32

def _sc_degree_body(dst_hbm, zeros_hbm, ones_hbm, out_hbm, acc, onesv, dstv):
    c = lax.axis_index("c")
    s = lax.axis_index("s")
    r0 = s * ROWS_PER_TILE
    pltpu.sync_copy(zeros_hbm, acc.at[pl.ds(r0, ROWS_PER_TILE)])
    pltpu.sync_copy(ones_hbm, onesv)
    pltpu.sync_copy(dst_hbm.at[c, s], dstv)
    plsc.subcore_barrier()

    def step(j, carry):
        pltpu.sync_copy(onesv, acc.at[dstv.at[j]], add=True)
        return carry

    lax.fori_loop(0, BLOCKS_A, step, 0)
    plsc.subcore_barrier()

    @pl.when(s < OUT_TILES)
    def _():
        pltpu.sync_copy(acc.at[pl.ds(s * OUT_ROWS, OUT_ROWS)],
                        out_hbm.at[c, s])


def _sc_gather_scatter_body(g_hbm, src_hbm, dst_hbm, zeros_hbm, out_hbm,
                            acc, rows0, rows1, srcv, dstv,
                            gsem0, gsem1, ssem0, ssem1):
    c = lax.axis_index("c")
    s = lax.axis_index("s")
    r0 = s * ROWS_PER_TILE
    pltpu.sync_copy(zeros_hbm, acc.at[pl.ds(r0, ROWS_PER_TILE)])
    plsc.subcore_barrier()

    rows = (rows0, rows1)
    gsem = (gsem0, gsem1)
    ssem = (ssem0, ssem1)

    def chunk(q, carry):
        pltpu.sync_copy(src_hbm.at[c, s, pl.ds(q * CHUNK_C, CHUNK_C)], srcv)
        pltpu.sync_copy(dst_hbm.at[s, pl.ds(q * CHUNK_C, CHUNK_C)], dstv)
        hg = [None] * CHUNK_C
        hs = [None] * CHUNK_C
        hg[0] = pltpu.async_copy(g_hbm.at[srcv.at[0]], rows[0], gsem[0])
        for j in range(CHUNK_C):
            b = j % 2
            hg[j].wait()
            hs[j] = pltpu.async_copy(rows[b], acc.at[dstv.at[j]], ssem[b],
                                     add=True)
            if j + 1 < CHUNK_C:
                if j >= 1:
                    hs[j - 1].wait()
                hg[j + 1] = pltpu.async_copy(g_hbm.at[srcv.at[j + 1]],
                                             rows[1 - b], gsem[1 - b])
        hs[CHUNK_C - 2].wait()
        hs[CHUNK_C - 1].wait()
        return carry

    lax.fori_loop(0, BLOCKS_C // CHUNK_C, chunk, 0)
    plsc.subcore_barrier()

    @pl.when(s < OUT_TILES)
    def _():
        pltpu.sync_copy(acc.at[pl.ds(s * OUT_ROWS, OUT_ROWS)],
                        out_hbm.at[c, s])


@functools.lru_cache(maxsize=None)
def _sc_kernels():
    mesh = plsc.VectorSubcoreMesh(core_axis_name="c", subcore_axis_name="s")
    sc_degree = pl.kernel(
        _sc_degree_body,
        mesh=mesh,
        out_type=jax.ShapeDtypeStruct((NC, OUT_TILES, OUT_ROWS, D),
                                      jnp.float32),
        scratch_types=[
            pltpu.VMEM_SHARED((N, D), jnp.float32),
            pltpu.VMEM((K, D), jnp.float32),
            pltpu.VMEM((BLOCKS_A, K), jnp.int32),
        ],
    )
    sc_gather_scatter = pl.kernel(
        _sc_gather_scatter_body,
        mesh=mesh,
        out_type=jax.ShapeDtypeStruct((NC, OUT_TILES, OUT_ROWS, D),
                                      jnp.float32),
        scratch_types=[
            pltpu.VMEM_SHARED((N, D), jnp.float32),
            pltpu.VMEM((K, D), jnp.float32),
            pltpu.VMEM((K, D), jnp.float32),
            pltpu.VMEM((CHUNK_C, K), jnp.int32),
            pltpu.VMEM((CHUNK_C, K), jnp.int32),
            pltpu.SemaphoreType.DMA,
            pltpu.SemaphoreType.DMA,
            pltpu.SemaphoreType.DMA,
            pltpu.SemaphoreType.DMA,
        ],
    )
    return sc_degree, sc_gather_scatter


def _prep_body(xs_ref, xt_ref, w1_ref, w2_ref, degp_ref, g_ref):
    deg = degp_ref[0][:, 0:1] + degp_ref[1][:, 0:1] + 1.0
    dinv = lax.rsqrt(deg)
    x = xs_ref[...]
    h1 = lax.dot_general(x, w1_ref[...], (((1,), (1,)), ((), ())),
                         preferred_element_type=jnp.float32)
    h2 = lax.dot_general(x + xt_ref[...], w2_ref[...], (((1,), (1,)), ((), ())),
                         preferred_element_type=jnp.float32)
    g_ref[0] = h1 * dinv
    g_ref[1] = h2 * dinv


def _tc_prepare(xs, xt, W1, W2, degp):
    bn = 1000
    grid = (N // bn,)
    return pl.pallas_call(
        _prep_body,
        grid=grid,
        in_specs=[
            pl.BlockSpec((bn, D), lambda i: (i, 0)),
            pl.BlockSpec((bn, D), lambda i: (i, 0)),
            pl.BlockSpec((D, D), lambda i: (0, 0)),
            pl.BlockSpec((D, D), lambda i: (0, 0)),
            pl.BlockSpec((NC, bn, D), lambda i: (0, i, 0)),
        ],
        out_specs=pl.BlockSpec((NC, bn, D), lambda i: (0, i, 0)),
        out_shape=jax.ShapeDtypeStruct((NC, N, D), jnp.float32),
    )(xs, xt, W1, W2, degp)


def _final_body(acc_ref, g_ref, degp_ref, xs_ref, xto_ref, b1_ref, b2_ref,
                embs_ref, etf_ref, cs_ref, ct_ref, *, ngrid):
    deg = degp_ref[0][:, 0:1] + degp_ref[1][:, 0:1] + 1.0
    dinv = lax.rsqrt(deg)
    xs = xs_ref[...]
    emb_s = dinv * (acc_ref[0] + g_ref[0]) + b1_ref[...] + xs
    etn = dinv * (acc_ref[1] + g_ref[1]) + b2_ref[...] + xs
    embs_ref[...] = emb_s
    xto = xto_ref[...]
    etf_ref[...] = xto + etn[None, :, :]

    cs_part = jnp.sum(emb_s, axis=0, keepdims=True)
    ct_part = (jnp.sum(xto, axis=(0, 1))[None, :] * (1.0 / T)
               + jnp.sum(etn, axis=0, keepdims=True))
    i = pl.program_id(0)

    @pl.when(i == 0)
    def _():
        cs_ref[...] = jnp.zeros_like(cs_ref)
        ct_ref[...] = jnp.zeros_like(ct_ref)

    cs_ref[...] += cs_part
    ct_ref[...] += ct_part

    @pl.when(i == ngrid - 1)
    def _():
        cs_ref[...] *= 1.0 / N
        ct_ref[...] *= 1.0 / N


def _tc_finalize(acc, g, degp, xs, xto, b1, b2):
    bn = 200
    ngrid = N // bn
    return pl.pallas_call(
        functools.partial(_final_body, ngrid=ngrid),
        grid=(ngrid,),
        in_specs=[
            pl.BlockSpec((NC, bn, D), lambda i: (0, i, 0)),
            pl.BlockSpec((NC, bn, D), lambda i: (0, i, 0)),
            pl.BlockSpec((NC, bn, D), lambda i: (0, i, 0)),
            pl.BlockSpec((bn, D), lambda i: (i, 0)),
            pl.BlockSpec((T, bn, D), lambda i: (0, i, 0)),
            pl.BlockSpec((1, D), lambda i: (0, 0)),
            pl.BlockSpec((1, D), lambda i: (0, 0)),
        ],
        out_specs=[
            pl.BlockSpec((bn, D), lambda i: (i, 0)),
            pl.BlockSpec((T, bn, D), lambda i: (0, i, 0)),
            pl.BlockSpec((1, D), lambda i: (0, 0)),
            pl.BlockSpec((1, D), lambda i: (0, 0)),
        ],
        out_shape=[
            jax.ShapeDtypeStruct((N, D), jnp.float32),
            jax.ShapeDtypeStruct((T, N, D), jnp.float32),
            jax.ShapeDtypeStruct((1, D), jnp.float32),
            jax.ShapeDtypeStruct((1, D), jnp.float32),
        ],
    )(acc, g, degp, xs, xto, b1, b2)


def kernel(x_sentance, x_token, x_token_ori, edge_index, W1, b1, W2, b2):
    src = edge_index[0]
    dst = edge_index[1]

    dstA = dst.reshape(NC, NS, BLOCKS_A, K)
    dstC = dst.reshape(NS, BLOCKS_C, K)
    srcC = src.reshape(NS, BLOCKS_C, K)
    srcC2 = jnp.stack([srcC, srcC + N])

    zerosD = jnp.zeros((ROWS_PER_TILE, D), jnp.float32)
    onesD = jnp.ones((K, D), jnp.float32)

    sc_degree, sc_gather_scatter = _sc_kernels()
    degp = sc_degree(dstA, zerosD, onesD).reshape(NC, N, D)
    g = _tc_prepare(x_sentance, x_token, W1, W2, degp)
    acc = sc_gather_scatter(g.reshape(NC * N, D), srcC2, dstC,
                            zerosD).reshape(NC, N, D)

    emb_s, etf_t, cs, ct = _tc_finalize(
        acc, g, degp, x_sentance, jnp.transpose(x_token_ori, (1, 0, 2)),
        b1.reshape(1, D), b2.reshape(1, D))
    return (emb_s, jnp.transpose(etf_t, (1, 0, 2)),
            cs.reshape(D), ct.reshape(D))


# speedup vs baseline: 20.6028x; 1.0224x over previous
import functools

import jax
import jax.numpy as jnp
from jax import lax
from jax.experimental import pallas as pl
from jax.experimental.pallas import tpu as pltpu
from jax.experimental.pallas import tpu_sc as plsc

N = 10000
E = 320000
D = 128
T = 20

NC = 2
NS = 16
ROWS_PER_TILE = N // NS
OUT_TILES = 10
OUT_ROWS = N // OUT_TILES
K = 125
BLOCKS_A = E // (NC * NS) // K
BLOCKS_C = E // NS // K
CHUNK_C = 32

def _sc_degree_body(dst_hbm, zeros_hbm, ones_hbm, out_hbm, acc, onesv, dstv):
    c = lax.axis_index("c")
    s = lax.axis_index("s")
    r0 = s * ROWS_PER_TILE
    pltpu.sync_copy(zeros_hbm, acc.at[pl.ds(r0, ROWS_PER_TILE)])
    pltpu.sync_copy(ones_hbm, onesv)
    pltpu.sync_copy(dst_hbm.at[c, s], dstv)
    plsc.subcore_barrier()

    def step(j, carry):
        pltpu.sync_copy(onesv, acc.at[dstv.at[j]], add=True)
        return carry

    lax.fori_loop(0, BLOCKS_A, step, 0)
    plsc.subcore_barrier()

    @pl.when(s < OUT_TILES)
    def _():
        pltpu.sync_copy(acc.at[pl.ds(s * OUT_ROWS, OUT_ROWS)],
                        out_hbm.at[c, s])


def _sc_gather_scatter_body(g_hbm, src_hbm, dst_hbm, zeros_hbm, out_hbm,
                            acc, rows0, rows1, srcv, dstv,
                            gsem0, gsem1, ssem0, ssem1):
    c = lax.axis_index("c")
    s = lax.axis_index("s")
    r0 = s * ROWS_PER_TILE
    pltpu.sync_copy(zeros_hbm, acc.at[pl.ds(r0, ROWS_PER_TILE)])
    plsc.subcore_barrier()

    rows = (rows0, rows1)
    gsem = (gsem0, gsem1)
    ssem = (ssem0, ssem1)

    def chunk(q, carry):
        pltpu.sync_copy(src_hbm.at[c, s, pl.ds(q * CHUNK_C, CHUNK_C)], srcv)
        pltpu.sync_copy(dst_hbm.at[s, pl.ds(q * CHUNK_C, CHUNK_C)], dstv)
        hg = [None] * CHUNK_C
        hs = [None] * CHUNK_C
        hg[0] = pltpu.async_copy(g_hbm.at[srcv.at[0]], rows[0], gsem[0])
        for j in range(CHUNK_C):
            b = j % 2
            hg[j].wait()
            hs[j] = pltpu.async_copy(rows[b], acc.at[dstv.at[j]], ssem[b],
                                     add=True)
            if j + 1 < CHUNK_C:
                if j >= 1:
                    hs[j - 1].wait()
                hg[j + 1] = pltpu.async_copy(g_hbm.at[srcv.at[j + 1]],
                                             rows[1 - b], gsem[1 - b])
        hs[CHUNK_C - 2].wait()
        hs[CHUNK_C - 1].wait()
        return carry

    lax.fori_loop(0, BLOCKS_C // CHUNK_C, chunk, 0)
    plsc.subcore_barrier()

    @pl.when(s < OUT_TILES)
    def _():
        pltpu.sync_copy(acc.at[pl.ds(s * OUT_ROWS, OUT_ROWS)],
                        out_hbm.at[c, s])


@functools.lru_cache(maxsize=None)
def _sc_kernels():
    mesh = plsc.VectorSubcoreMesh(core_axis_name="c", subcore_axis_name="s")
    sc_degree = pl.kernel(
        _sc_degree_body,
        mesh=mesh,
        out_type=jax.ShapeDtypeStruct((NC, OUT_TILES, OUT_ROWS, D),
                                      jnp.float32),
        scratch_types=[
            pltpu.VMEM_SHARED((N, D), jnp.float32),
            pltpu.VMEM((K, D), jnp.float32),
            pltpu.VMEM((BLOCKS_A, K), jnp.int32),
        ],
    )
    sc_gather_scatter = pl.kernel(
        _sc_gather_scatter_body,
        mesh=mesh,
        out_type=jax.ShapeDtypeStruct((NC, OUT_TILES, OUT_ROWS, D),
                                      jnp.float32),
        scratch_types=[
            pltpu.VMEM_SHARED((N, D), jnp.float32),
            pltpu.VMEM((K, D), jnp.float32),
            pltpu.VMEM((K, D), jnp.float32),
            pltpu.VMEM((CHUNK_C, K), jnp.int32),
            pltpu.VMEM((CHUNK_C, K), jnp.int32),
            pltpu.SemaphoreType.DMA,
            pltpu.SemaphoreType.DMA,
            pltpu.SemaphoreType.DMA,
            pltpu.SemaphoreType.DMA,
        ],
    )
    return sc_degree, sc_gather_scatter


def _prep_body(xs_ref, xt_ref, w1_ref, w2_ref, degp_ref, g_ref):
    deg = degp_ref[0][:, 0:1] + degp_ref[1][:, 0:1] + 1.0
    dinv = lax.rsqrt(deg)
    x = xs_ref[...]
    h1 = lax.dot_general(x, w1_ref[...], (((1,), (1,)), ((), ())),
                         preferred_element_type=jnp.float32)
    h2 = lax.dot_general(x + xt_ref[...], w2_ref[...], (((1,), (1,)), ((), ())),
                         preferred_element_type=jnp.float32)
    g_ref[0] = h1 * dinv
    g_ref[1] = h2 * dinv


def _tc_prepare(xs, xt, W1, W2, degp):
    bn = 1000
    grid = (N // bn,)
    return pl.pallas_call(
        _prep_body,
        grid=grid,
        in_specs=[
            pl.BlockSpec((bn, D), lambda i: (i, 0)),
            pl.BlockSpec((bn, D), lambda i: (i, 0)),
            pl.BlockSpec((D, D), lambda i: (0, 0)),
            pl.BlockSpec((D, D), lambda i: (0, 0)),
            pl.BlockSpec((NC, bn, D), lambda i: (0, i, 0)),
        ],
        out_specs=pl.BlockSpec((NC, bn, D), lambda i: (0, i, 0)),
        out_shape=jax.ShapeDtypeStruct((NC, N, D), jnp.float32),
    )(xs, xt, W1, W2, degp)


def _final_body(acc_ref, g_ref, degp_ref, xs_ref, xto_ref, b1_ref, b2_ref,
                embs_ref, etf_ref, cs_ref, ct_ref, *, ngrid):
    deg = degp_ref[0][:, 0:1] + degp_ref[1][:, 0:1] + 1.0
    dinv = lax.rsqrt(deg)
    xs = xs_ref[...]
    emb_s = dinv * (acc_ref[0] + g_ref[0]) + b1_ref[...] + xs
    etn = dinv * (acc_ref[1] + g_ref[1]) + b2_ref[...] + xs
    embs_ref[...] = emb_s
    xto = xto_ref[...]
    etf_ref[...] = xto + etn[None, :, :]

    cs_part = jnp.sum(emb_s, axis=0, keepdims=True)
    ct_part = (jnp.sum(xto, axis=(0, 1))[None, :] * (1.0 / T)
               + jnp.sum(etn, axis=0, keepdims=True))
    i = pl.program_id(0)

    @pl.when(i == 0)
    def _():
        cs_ref[...] = jnp.zeros_like(cs_ref)
        ct_ref[...] = jnp.zeros_like(ct_ref)

    cs_ref[...] += cs_part
    ct_ref[...] += ct_part

    @pl.when(i == ngrid - 1)
    def _():
        cs_ref[...] *= 1.0 / N
        ct_ref[...] *= 1.0 / N


def _tc_finalize(acc, g, degp, xs, xto, b1, b2):
    bn = 400
    ngrid = N // bn
    return pl.pallas_call(
        functools.partial(_final_body, ngrid=ngrid),
        grid=(ngrid,),
        in_specs=[
            pl.BlockSpec((NC, bn, D), lambda i: (0, i, 0)),
            pl.BlockSpec((NC, bn, D), lambda i: (0, i, 0)),
            pl.BlockSpec((NC, bn, D), lambda i: (0, i, 0)),
            pl.BlockSpec((bn, D), lambda i: (i, 0)),
            pl.BlockSpec((T, bn, D), lambda i: (0, i, 0)),
            pl.BlockSpec((1, D), lambda i: (0, 0)),
            pl.BlockSpec((1, D), lambda i: (0, 0)),
        ],
        out_specs=[
            pl.BlockSpec((bn, D), lambda i: (i, 0)),
            pl.BlockSpec((T, bn, D), lambda i: (0, i, 0)),
            pl.BlockSpec((1, D), lambda i: (0, 0)),
            pl.BlockSpec((1, D), lambda i: (0, 0)),
        ],
        out_shape=[
            jax.ShapeDtypeStruct((N, D), jnp.float32),
            jax.ShapeDtypeStruct((T, N, D), jnp.float32),
            jax.ShapeDtypeStruct((1, D), jnp.float32),
            jax.ShapeDtypeStruct((1, D), jnp.float32),
        ],
    )(acc, g, degp, xs, xto, b1, b2)


def kernel(x_sentance, x_token, x_token_ori, edge_index, W1, b1, W2, b2):
    src = edge_index[0]
    dst = edge_index[1]

    dstA = dst.reshape(NC, NS, BLOCKS_A, K)
    dstC = dst.reshape(NS, BLOCKS_C, K)
    srcC = src.reshape(NS, BLOCKS_C, K)
    srcC2 = jnp.stack([srcC, srcC + N])

    zerosD = jnp.zeros((ROWS_PER_TILE, D), jnp.float32)
    onesD = jnp.ones((K, D), jnp.float32)

    sc_degree, sc_gather_scatter = _sc_kernels()
    degp = sc_degree(dstA, zerosD, onesD).reshape(NC, N, D)
    g = _tc_prepare(x_sentance, x_token, W1, W2, degp)
    acc = sc_gather_scatter(g.reshape(NC * N, D), srcC2, dstC,
                            zerosD).reshape(NC, N, D)

    emb_s, etf_t, cs, ct = _tc_finalize(
        acc, g, degp, x_sentance, jnp.transpose(x_token_ori, (1, 0, 2)),
        b1.reshape(1, D), b2.reshape(1, D))
    return (emb_s, jnp.transpose(etf_t, (1, 0, 2)),
            cs.reshape(D), ct.reshape(D))
